# baseline (speedup 1.0000x reference)
import functools

import jax
import jax.numpy as jnp
import numpy as np
from jax import lax
from jax.experimental import pallas as pl
from jax.experimental.pallas import tpu as pltpu
from jax.experimental.pallas import tpu_sc as plsc

DIM = 256
HEADS = 8
DH = DIM // HEADS
LEVELS = 4
ASP = 8
NPTS = 4
NCAM = 6
NQ = 2500
_SHAPES = np.array([[60, 100], [30, 50], [15, 25], [8, 13]], dtype=np.int64)
NV = int((_SHAPES[:, 0] * _SHAPES[:, 1]).sum())
_LSI = np.concatenate(
    [np.zeros(1, dtype=np.int64), np.cumsum(_SHAPES[:, 0] * _SHAPES[:, 1])[:-1]]
)

QPT = LEVELS * ASP
WPT = QPT * 4
ROW = 4 * DH
TASKS = NCAM * NQ * HEADS
NWORKERS = 32
TPB = 8
NCHUNKS = TASKS // TPB
_EXTRA = NCHUNKS % NWORKERS


def _sc_gather_reduce(idx, wgt, table):
    mesh = plsc.VectorSubcoreMesh(core_axis_name="c", subcore_axis_name="s")

    @functools.partial(
        pl.kernel,
        mesh=mesh,
        out_type=jax.ShapeDtypeStruct((TASKS, DH), jnp.float32),
        scratch_types=[
            pltpu.VMEM((2, TPB, QPT), jnp.int32),
            pltpu.VMEM((2, TPB, WPT), jnp.float32),
            pltpu.VMEM((2, TPB, QPT, ROW), jnp.float32),
            pltpu.VMEM((TPB, DH), jnp.float32),
            pltpu.SemaphoreType.DMA,
            pltpu.SemaphoreType.DMA,
        ],
    )
    def body(idx_hbm, wgt_hbm, table_hbm, out_hbm, idx_v, wgt_v, rows_v,
             out_v, sem0, sem1):
        wid = lax.axis_index("s") * 2 + lax.axis_index("c")
        nchunks = jnp.where(wid < _EXTRA, NCHUNKS // NWORKERS + 1,
                            NCHUNKS // NWORKERS)
        sems = [sem0, sem1]

        def stage_fire(buf, c):
            task0 = (wid + c * NWORKERS) * TPB
            pltpu.sync_copy(idx_hbm.at[pl.ds(task0, TPB)], idx_v.at[buf])
            pltpu.sync_copy(wgt_hbm.at[pl.ds(task0, TPB)], wgt_v.at[buf])
            for t in range(TPB):
                pltpu.async_copy(table_hbm.at[idx_v.at[buf, t]],
                                 rows_v.at[buf, t], sems[buf])

        def drain_compute_store(buf, c):
            task0 = (wid + c * NWORKERS) * TPB
            for t in range(TPB):
                pltpu.make_async_copy(table_hbm.at[idx_v.at[buf, t]],
                                      rows_v.at[buf, t], sems[buf]).wait()

            def grp_body(qg, accs):
                new = list(accs)
                for t in range(TPB):
                    w16 = wgt_v[buf, t, pl.ds(qg * 16, 16)]
                    for pp in range(4):
                        q = qg * 4 + pp
                        for c4 in range(4):
                            w = w16[pp * 4 + c4]
                            lo = rows_v[buf, t, q, pl.ds(c4 * DH, 16)]
                            hi = rows_v[buf, t, q, pl.ds(c4 * DH + 16, 16)]
                            new[2 * t] = new[2 * t] + w * lo
                            new[2 * t + 1] = new[2 * t + 1] + w * hi
                return tuple(new)

            zeros = tuple(jnp.zeros((16,), jnp.float32)
                          for _ in range(2 * TPB))
            accs = lax.fori_loop(0, QPT // 4, grp_body, zeros, unroll=True)
            for t in range(TPB):
                out_v[t, pl.ds(0, 16)] = accs[2 * t]
                out_v[t, pl.ds(16, 16)] = accs[2 * t + 1]
            pltpu.sync_copy(out_v.at[...], out_hbm.at[pl.ds(task0, TPB)])

        stage_fire(0, 0)

        def pair_body(m, _):
            c0 = 2 * m
            c1 = 2 * m + 1

            @pl.when(c1 < nchunks)
            def _():
                stage_fire(1, c1)

            drain_compute_store(0, c0)

            @pl.when(c0 + 2 < nchunks)
            def _():
                stage_fire(0, c0 + 2)

            @pl.when(c1 < nchunks)
            def _():
                drain_compute_store(1, c1)

            return ()

        lax.fori_loop(0, (nchunks + 1) // 2, pair_body, (), unroll=False)

    return body(idx, wgt, table)


def _matmul_bias_kernel(x_ref, w_ref, b_ref, o_ref):
    acc = jnp.dot(x_ref[...], w_ref[...], preferred_element_type=jnp.float32)
    o_ref[...] = acc + b_ref[...]


def _matmul_bias(x, W, b):
    m = x.shape[0]
    k = x.shape[1]
    n = W.shape[1]
    pad = (-m) % 128
    xp = jnp.pad(x, ((0, pad), (0, 0)))
    mp = m + pad
    out = pl.pallas_call(
        _matmul_bias_kernel,
        grid=(mp // 128,),
        in_specs=[
            pl.BlockSpec((128, k), lambda i: (i, 0)),
            pl.BlockSpec((k, n), lambda i: (0, 0)),
            pl.BlockSpec((1, n), lambda i: (0, 0)),
        ],
        out_specs=pl.BlockSpec((128, n), lambda i: (i, 0)),
        out_shape=jax.ShapeDtypeStruct((mp, n), jnp.float32),
    )(xp, W, b.reshape(1, n))
    return out[:m]


def _out_proj_kernel(slots_ref, resid_ref, w_ref, b_ref, o_ref):
    acc = jnp.dot(slots_ref[...], w_ref[...], preferred_element_type=jnp.float32)
    o_ref[...] = acc + b_ref[...] + resid_ref[...]


def _out_projection(slots, resid, W_out, b_out):
    m = slots.shape[0]
    pad = (-m) % 128
    sp = jnp.pad(slots, ((0, pad), (0, 0)))
    rp = jnp.pad(resid, ((0, pad), (0, 0)))
    mp = m + pad
    out = pl.pallas_call(
        _out_proj_kernel,
        grid=(mp // 128,),
        in_specs=[
            pl.BlockSpec((128, DIM), lambda i: (i, 0)),
            pl.BlockSpec((128, DIM), lambda i: (i, 0)),
            pl.BlockSpec((DIM, DIM), lambda i: (0, 0)),
            pl.BlockSpec((1, DIM), lambda i: (0, 0)),
        ],
        out_specs=pl.BlockSpec((128, DIM), lambda i: (i, 0)),
        out_shape=jax.ShapeDtypeStruct((mp, DIM), jnp.float32),
    )(sp, rp, W_out, b_out.reshape(1, DIM))
    return out[:m]


def _pack_kernel(v_ref, o_ref):
    for lvl in range(LEVELS):
        H = int(_SHAPES[lvl][0])
        W = int(_SHAPES[lvl][1])
        lsi = int(_LSI[lvl])
        x = v_ref[0, 0, pl.ds(lsi, H * W), :].reshape(H, W, DH)
        zrow = jnp.zeros((1, W, DH), jnp.float32)
        zcol = jnp.zeros((H, 1, DH), jnp.float32)
        xs = jnp.concatenate([x[:, 1:, :], zcol], axis=1)
        xd = jnp.concatenate([x[1:, :, :], zrow], axis=0)
        xds = jnp.concatenate([xd[:, 1:, :], zcol], axis=1)
        t4 = jnp.concatenate([x, xs, xd, xds], axis=-1)
        o_ref[0, 0, pl.ds(lsi, H * W), :] = t4.reshape(H * W, ROW)


def _pack_corner_table(v_t):
    out = pl.pallas_call(
        _pack_kernel,
        grid=(NCAM, HEADS),
        in_specs=[pl.BlockSpec((1, 1, NV, DH), lambda c, h: (c, h, 0, 0))],
        out_specs=pl.BlockSpec((1, 1, NV, ROW), lambda c, h: (c, h, 0, 0)),
        out_shape=jax.ShapeDtypeStruct((NCAM, HEADS, NV, ROW), jnp.float32),
    )(v_t)
    return out.reshape(NCAM * HEADS * NV, ROW)


def kernel(query, key, value, ref_points, vox_mask, spatial_shapes,
           level_start_index, W_off, b_off, W_att, b_att, W_val, b_val,
           W_out, b_out):
    del key, level_start_index
    q2 = query[0]

    W_q = jnp.concatenate([W_off, W_att], axis=1)
    b_q = jnp.concatenate([b_off, b_att], axis=0)
    qp = _matmul_bias(q2, W_q, b_q)
    so = qp[:, : HEADS * LEVELS * ASP * 2].reshape(NQ, HEADS, LEVELS, 2, NPTS, 2)
    aw = jax.nn.softmax(
        qp[:, HEADS * LEVELS * ASP * 2:].reshape(NQ, HEADS, LEVELS * ASP), axis=-1)
    aw = aw.reshape(NQ, HEADS, LEVELS, ASP)

    v = _matmul_bias(value[:, :, 0, :].reshape(NCAM * NV, DIM), W_val, b_val)
    v_t = v.reshape(NCAM, NV, HEADS, DH).transpose(0, 2, 1, 3)
    table = _pack_corner_table(v_t)

    norm = np.stack([_SHAPES[:, 1], _SHAPES[:, 0]], -1).astype(np.float32)
    so_n = so / norm[None, None, :, None, None, :]
    rp = ref_points[:, 0]
    sl = rp[:, :, None, None, None, :, :] + so_n[None]
    sl = sl.reshape(NCAM, NQ, HEADS, LEVELS, ASP, 2)

    xy = sl * norm[None, None, None, :, None, :] - 0.5
    x, y = xy[..., 0], xy[..., 1]
    Wl = _SHAPES[:, 1].astype(np.float32)[None, None, None, :, None]
    Hl = _SHAPES[:, 0].astype(np.float32)[None, None, None, :, None]
    lsi = _LSI.astype(np.int32)[None, None, None, :, None]

    bx = jnp.clip(jnp.floor(x), 0, Wl - 1)
    by = jnp.clip(jnp.floor(y), 0, Hl - 1)
    bidx = lsi + by.astype(jnp.int32) * Wl.astype(jnp.int32) + bx.astype(jnp.int32)

    wgt_list = []
    for dy in (0, 1):
        for dx in (0, 1):
            cx = bx + dx
            ry = by + dy
            w = (jnp.maximum(0.0, 1.0 - jnp.abs(x - cx))
                 * jnp.maximum(0.0, 1.0 - jnp.abs(y - ry)))
            w = jnp.where((cx <= Wl - 1) & (ry <= Hl - 1), w, 0.0)
            wgt_list.append(w * aw[None])
    wgt4 = jnp.stack(wgt_list, axis=-1)

    cam_head = (
        jnp.arange(NCAM, dtype=jnp.int32)[:, None, None, None, None] * HEADS
        + jnp.arange(HEADS, dtype=jnp.int32)[None, None, :, None, None]
    )
    gidx = (cam_head * NV + bidx).reshape(TASKS, QPT)
    gwgt = wgt4.reshape(TASKS, WPT)

    out = _sc_gather_reduce(gidx, gwgt, table)
    out = out.reshape(NCAM, NQ, DIM)

    m = (vox_mask[:, 0].sum(-1) > 0)
    slots = jnp.sum(jnp.where(m[:, :, None], out, 0.0), axis=0)
    count = jnp.maximum(m.sum(0).astype(jnp.float32), 1.0)
    slots = slots / count[:, None]

    res = _out_projection(slots, q2, W_out, b_out)
    return res[None]


# speedup vs baseline: 1.3257x; 1.3257x over previous
import functools

import jax
import jax.numpy as jnp
import numpy as np
from jax import lax
from jax.experimental import pallas as pl
from jax.experimental.pallas import tpu as pltpu
from jax.experimental.pallas import tpu_sc as plsc

DIM = 256
HEADS = 8
DH = DIM // HEADS
LEVELS = 4
ASP = 8
NPTS = 4
NCAM = 6
NQ = 2500
_SHAPES = np.array([[60, 100], [30, 50], [15, 25], [8, 13]], dtype=np.int64)
NV = int((_SHAPES[:, 0] * _SHAPES[:, 1]).sum())
_LSI = np.concatenate(
    [np.zeros(1, dtype=np.int64), np.cumsum(_SHAPES[:, 0] * _SHAPES[:, 1])[:-1]]
)

QPT = LEVELS * ASP
WPT = QPT * 4
ROW = 4 * DH
TASKS = NCAM * NQ * HEADS
NWORKERS = 32
TPB = 8
NCHUNKS = TASKS // TPB
_EXTRA = NCHUNKS % NWORKERS


def _sc_gather_reduce(idx, wgt, table):
    mesh = plsc.VectorSubcoreMesh(core_axis_name="c", subcore_axis_name="s")

    @functools.partial(
        pl.kernel,
        mesh=mesh,
        out_type=jax.ShapeDtypeStruct((TASKS, DH), jnp.float32),
        scratch_types=[
            pltpu.VMEM((2, TPB, QPT), jnp.int32),
            pltpu.VMEM((2, TPB, WPT), jnp.float32),
            pltpu.VMEM((2, TPB, QPT, ROW), jnp.float32),
            pltpu.VMEM((2, TPB, DH), jnp.float32),
            pltpu.SemaphoreType.DMA,
            pltpu.SemaphoreType.DMA,
            pltpu.SemaphoreType.DMA,
            pltpu.SemaphoreType.DMA,
            pltpu.SemaphoreType.DMA,
            pltpu.SemaphoreType.DMA,
        ],
    )
    def body(idx_hbm, wgt_hbm, table_hbm, out_hbm, st_i, st_w, rows_v,
             out_v, sem_s0, sem_s1, sem_g0, sem_g1, sem_o0, sem_o1):
        wid = lax.axis_index("s") * 2 + lax.axis_index("c")
        nchunks = jnp.where(wid < _EXTRA, NCHUNKS // NWORKERS + 1,
                            NCHUNKS // NWORKERS)
        sem_s = [sem_s0, sem_s1]
        sem_g = [sem_g0, sem_g1]
        sem_o = [sem_o0, sem_o1]

        def stage(buf, c):
            task0 = (wid + c * NWORKERS) * TPB
            pltpu.async_copy(idx_hbm.at[pl.ds(task0, TPB)], st_i.at[buf],
                             sem_s[buf])
            pltpu.async_copy(wgt_hbm.at[pl.ds(task0, TPB)], st_w.at[buf],
                             sem_s[buf])

        def fire(buf, c):
            task0 = (wid + c * NWORKERS) * TPB
            pltpu.make_async_copy(idx_hbm.at[pl.ds(task0, TPB)],
                                  st_i.at[buf], sem_s[buf]).wait()
            pltpu.make_async_copy(wgt_hbm.at[pl.ds(task0, TPB)],
                                  st_w.at[buf], sem_s[buf]).wait()
            for t in range(TPB):
                pltpu.async_copy(
                    table_hbm.at[st_i.at[buf, t]],
                    rows_v.at[buf, t], sem_g[buf])

        def compute(buf, c, m):
            task0 = (wid + c * NWORKERS) * TPB
            for t in range(TPB):
                pltpu.make_async_copy(
                    table_hbm.at[st_i.at[buf, t]],
                    rows_v.at[buf, t], sem_g[buf]).wait()

            def grp_body(qg, accs):
                new = list(accs)
                for t in range(TPB):
                    w16 = st_w[buf, t, pl.ds(qg * 16, 16)]
                    for pp in range(4):
                        q = qg * 4 + pp
                        for c4 in range(4):
                            w = w16[pp * 4 + c4]
                            lo = rows_v[buf, t, q, pl.ds(c4 * DH, 16)]
                            hi = rows_v[buf, t, q, pl.ds(c4 * DH + 16, 16)]
                            new[2 * t] = new[2 * t] + w * lo
                            new[2 * t + 1] = new[2 * t + 1] + w * hi
                return tuple(new)

            zeros = tuple(jnp.zeros((16,), jnp.float32)
                          for _ in range(2 * TPB))
            accs = lax.fori_loop(0, QPT // 4, grp_body, zeros)

            @pl.when(m > 0)
            def _():
                pltpu.make_async_copy(out_v.at[buf],
                                      out_hbm.at[pl.ds(0, TPB)],
                                      sem_o[buf]).wait()
            for t in range(TPB):
                out_v[buf, t, pl.ds(0, 16)] = accs[2 * t]
                out_v[buf, t, pl.ds(16, 16)] = accs[2 * t + 1]
            pltpu.async_copy(out_v.at[buf], out_hbm.at[pl.ds(task0, TPB)],
                             sem_o[buf])

        stage(0, 0)
        fire(0, 0)
        stage(1, 1)

        def pair_body(m, _):
            c0 = 2 * m
            c1 = 2 * m + 1

            @pl.when(c1 < nchunks)
            def _():
                fire(1, c1)

            compute(0, c0, m)

            @pl.when(c0 + 2 < nchunks)
            def _():
                stage(0, c0 + 2)
                fire(0, c0 + 2)

            @pl.when(c1 < nchunks)
            def _():
                compute(1, c1, m)

            @pl.when(c1 + 2 < nchunks)
            def _():
                stage(1, c1 + 2)

            return ()

        lax.fori_loop(0, (nchunks + 1) // 2, pair_body, (), unroll=False)

        pltpu.make_async_copy(out_v.at[0], out_hbm.at[pl.ds(0, TPB)],
                              sem_o0).wait()
        pltpu.make_async_copy(out_v.at[1], out_hbm.at[pl.ds(0, TPB)],
                              sem_o1).wait()

    return body(idx, wgt, table)


def _matmul_bias_kernel(x_ref, w_ref, b_ref, o_ref):
    acc = jnp.dot(x_ref[...], w_ref[...], preferred_element_type=jnp.float32)
    o_ref[...] = acc + b_ref[...]


def _matmul_bias(x, W, b):
    m = x.shape[0]
    k = x.shape[1]
    n = W.shape[1]
    pad = (-m) % 128
    xp = jnp.pad(x, ((0, pad), (0, 0)))
    mp = m + pad
    out = pl.pallas_call(
        _matmul_bias_kernel,
        grid=(mp // 128,),
        in_specs=[
            pl.BlockSpec((128, k), lambda i: (i, 0)),
            pl.BlockSpec((k, n), lambda i: (0, 0)),
            pl.BlockSpec((1, n), lambda i: (0, 0)),
        ],
        out_specs=pl.BlockSpec((128, n), lambda i: (i, 0)),
        out_shape=jax.ShapeDtypeStruct((mp, n), jnp.float32),
    )(xp, W, b.reshape(1, n))
    return out[:m]


def _out_proj_kernel(slots_ref, resid_ref, w_ref, b_ref, o_ref):
    acc = jnp.dot(slots_ref[...], w_ref[...], preferred_element_type=jnp.float32)
    o_ref[...] = acc + b_ref[...] + resid_ref[...]


def _out_projection(slots, resid, W_out, b_out):
    m = slots.shape[0]
    pad = (-m) % 128
    sp = jnp.pad(slots, ((0, pad), (0, 0)))
    rp = jnp.pad(resid, ((0, pad), (0, 0)))
    mp = m + pad
    out = pl.pallas_call(
        _out_proj_kernel,
        grid=(mp // 128,),
        in_specs=[
            pl.BlockSpec((128, DIM), lambda i: (i, 0)),
            pl.BlockSpec((128, DIM), lambda i: (i, 0)),
            pl.BlockSpec((DIM, DIM), lambda i: (0, 0)),
            pl.BlockSpec((1, DIM), lambda i: (0, 0)),
        ],
        out_specs=pl.BlockSpec((128, DIM), lambda i: (i, 0)),
        out_shape=jax.ShapeDtypeStruct((mp, DIM), jnp.float32),
    )(sp, rp, W_out, b_out.reshape(1, DIM))
    return out[:m]


def _pack_kernel(v_ref, o_ref):
    for lvl in range(LEVELS):
        H = int(_SHAPES[lvl][0])
        W = int(_SHAPES[lvl][1])
        lsi = int(_LSI[lvl])
        x = v_ref[0, 0, pl.ds(lsi, H * W), :].reshape(H, W, DH)
        zrow = jnp.zeros((1, W, DH), jnp.float32)
        zcol = jnp.zeros((H, 1, DH), jnp.float32)
        xs = jnp.concatenate([x[:, 1:, :], zcol], axis=1)
        xd = jnp.concatenate([x[1:, :, :], zrow], axis=0)
        xds = jnp.concatenate([xd[:, 1:, :], zcol], axis=1)
        t4 = jnp.concatenate([x, xs, xd, xds], axis=-1)
        o_ref[0, 0, pl.ds(lsi, H * W), :] = t4.reshape(H * W, ROW)


def _pack_corner_table(v_t):
    out = pl.pallas_call(
        _pack_kernel,
        grid=(NCAM, HEADS),
        in_specs=[pl.BlockSpec((1, 1, NV, DH), lambda c, h: (c, h, 0, 0))],
        out_specs=pl.BlockSpec((1, 1, NV, ROW), lambda c, h: (c, h, 0, 0)),
        out_shape=jax.ShapeDtypeStruct((NCAM, HEADS, NV, ROW), jnp.float32),
    )(v_t)
    return out.reshape(NCAM * HEADS * NV, ROW)


def kernel(query, key, value, ref_points, vox_mask, spatial_shapes,
           level_start_index, W_off, b_off, W_att, b_att, W_val, b_val,
           W_out, b_out):
    del key, level_start_index
    q2 = query[0]

    W_q = jnp.concatenate([W_off, W_att], axis=1)
    b_q = jnp.concatenate([b_off, b_att], axis=0)
    qp = _matmul_bias(q2, W_q, b_q)
    so = qp[:, : HEADS * LEVELS * ASP * 2].reshape(NQ, HEADS, LEVELS, 2, NPTS, 2)
    aw = jax.nn.softmax(
        qp[:, HEADS * LEVELS * ASP * 2:].reshape(NQ, HEADS, LEVELS * ASP), axis=-1)
    aw = aw.reshape(NQ, HEADS, LEVELS, ASP)

    v = _matmul_bias(value[:, :, 0, :].reshape(NCAM * NV, DIM), W_val, b_val)
    v_t = v.reshape(NCAM, NV, HEADS, DH).transpose(0, 2, 1, 3)
    table = _pack_corner_table(v_t)

    norm = np.stack([_SHAPES[:, 1], _SHAPES[:, 0]], -1).astype(np.float32)
    so_n = so / norm[None, None, :, None, None, :]
    rp = ref_points[:, 0]
    sl = rp[:, :, None, None, None, :, :] + so_n[None]
    sl = sl.reshape(NCAM, NQ, HEADS, LEVELS, ASP, 2)

    xy = sl * norm[None, None, None, :, None, :] - 0.5
    x, y = xy[..., 0], xy[..., 1]
    Wl = _SHAPES[:, 1].astype(np.float32)[None, None, None, :, None]
    Hl = _SHAPES[:, 0].astype(np.float32)[None, None, None, :, None]
    lsi = _LSI.astype(np.int32)[None, None, None, :, None]

    bx = jnp.clip(jnp.floor(x), 0, Wl - 1)
    by = jnp.clip(jnp.floor(y), 0, Hl - 1)
    bidx = lsi + by.astype(jnp.int32) * Wl.astype(jnp.int32) + bx.astype(jnp.int32)

    wgt_list = []
    for dy in (0, 1):
        for dx in (0, 1):
            cx = bx + dx
            ry = by + dy
            w = (jnp.maximum(0.0, 1.0 - jnp.abs(x - cx))
                 * jnp.maximum(0.0, 1.0 - jnp.abs(y - ry)))
            w = jnp.where((cx <= Wl - 1) & (ry <= Hl - 1), w, 0.0)
            wgt_list.append(w * aw[None])
    wgt4 = jnp.stack(wgt_list, axis=-1)

    cam_head = (
        jnp.arange(NCAM, dtype=jnp.int32)[:, None, None, None, None] * HEADS
        + jnp.arange(HEADS, dtype=jnp.int32)[None, None, :, None, None]
    )
    gidx = (cam_head * NV + bidx).reshape(TASKS, QPT)
    gwgt = wgt4.reshape(TASKS, WPT)

    out = _sc_gather_reduce(gidx, gwgt, table)
    out = out.reshape(NCAM, NQ, DIM)

    m = (vox_mask[:, 0].sum(-1) > 0)
    slots = jnp.sum(jnp.where(m[:, :, None], out, 0.0), axis=0)
    count = jnp.maximum(m.sum(0).astype(jnp.float32), 1.0)
    slots = slots / count[:, None]

    res = _out_projection(slots, q2, W_out, b_out)
    return res[None]
